# Initial kernel scaffold; baseline (speedup 1.0000x reference)
import jax
import jax.numpy as jnp
from jax.experimental import pallas as pl


def kernel(obs, h_state, hist, bias, ff_w, rec_w, ff_src, ff_dst, rec_src, rec_dst, rec_delay):
    raise NotImplementedError("write your pallas kernel here")


# speedup vs baseline: 225.8167x; 225.8167x over previous
import functools
import jax
import jax.numpy as jnp
from jax import lax
from jax.experimental import pallas as pl
from jax.experimental.pallas import tpu as pltpu
from jax.experimental.pallas import tpu_sc as plsc

N = 100000
N_IN = 512
N_OUT = 256
E_FF = 6400000
E_REC = 640000
D = 4
THRESH = N - N_OUT

NW = 32
FF_PER = E_FF // NW
REC_PER = E_REC // NW
C = 4000
FF_CHUNKS = FF_PER // C
REC_CHUNKS = REC_PER // C
M = 8192
L = 16


@functools.partial(
    pl.kernel,
    mesh=plsc.VectorSubcoreMesh(core_axis_name="c", subcore_axis_name="s"),
    compiler_params=pltpu.CompilerParams(needs_layout_passes=False),
    out_type=jax.ShapeDtypeStruct((NW, N_OUT), jnp.float32),
    scratch_types=[
        pltpu.VMEM((N,), jnp.float32),
        pltpu.VMEM((C,), jnp.int32),
        pltpu.VMEM((M,), jnp.int32),
        pltpu.VMEM((L * N_OUT,), jnp.float32),
        pltpu.VMEM((L,), jnp.int32),
        pltpu.VMEM((L,), jnp.float32),
        pltpu.VMEM((L,), jnp.int32),
        pltpu.VMEM((L,), jnp.float32),
        pltpu.VMEM((N_OUT,), jnp.float32),
    ],
)
def _sc_partials(hprev_hbm, hist_hbm, ffsrc_hbm, ffdst_hbm, ffw_hbm,
                 recsrc_hbm, recdst_hbm, recdelay_hbm, recw_hbm,
                 out_hbm,
                 hprev_v, dstbuf_v, enc_v, acc_v, src16_v, w16_v, d16_v,
                 h16_v, outbuf_v):
    wid = lax.axis_index("s") * 2 + lax.axis_index("c")
    iota = lax.iota(jnp.int32, L)
    ones = jnp.ones((L,), jnp.int32)

    pltpu.sync_copy(hprev_hbm, hprev_v)

    def zacc(i, _):
        acc_v[pl.ds(i * L, L)] = jnp.zeros((L,), jnp.float32)
        return 0
    lax.fori_loop(0, (L * N_OUT) // L, zacc, 0)

    def scan_phase(dst_hbm, base, nchunks):
        def chunk_body(c_i, n):
            pltpu.sync_copy(dst_hbm.at[pl.ds(base + c_i * C, C)], dstbuf_v)

            def vec_body(i, n):
                d = dstbuf_v[pl.ds(i * L, L)]
                m = d >= THRESH
                cs = plsc.cumsum(m.astype(jnp.int32))
                pos = jnp.minimum(n + cs, M - 1)
                gi = (c_i * C + i * L) + iota
                enc = (gi << 8) | (d - THRESH)
                plsc.store_scatter(enc_v, [pos], enc, mask=m)
                return n + jnp.max(cs)

            return lax.fori_loop(0, C // L, vec_body, n)

        n = lax.fori_loop(0, nchunks, chunk_body, jnp.int32(-1))
        n_tot = n + 1
        return jnp.minimum(n_tot, M)

    def ff_groups(n_tot):
        ngroups = (n_tot + (L - 1)) >> 4

        def g_body(g, _):
            e = enc_v[pl.ds(g * L, L)]
            valid = (g * L + iota) < n_tot
            gi = jnp.where(valid, e >> 8, 0)
            dl = e & (N_OUT - 1)
            gidx = wid * FF_PER + gi
            pltpu.sync_copy(ffsrc_hbm.at[gidx], src16_v)
            pltpu.sync_copy(ffw_hbm.at[gidx], w16_v)
            h = plsc.load_gather(hprev_v, [src16_v[...]])
            val = jnp.where(valid, w16_v[...] * h, 0.0)
            plsc.addupdate_scatter(acc_v, [(iota << 8) | dl], val)
            return 0

        lax.fori_loop(0, ngroups, g_body, 0)

    def rec_groups(n_tot):
        ngroups = (n_tot + (L - 1)) >> 4

        def g_body(g, _):
            e = enc_v[pl.ds(g * L, L)]
            valid = (g * L + iota) < n_tot
            gi = jnp.where(valid, e >> 8, 0)
            dl = e & (N_OUT - 1)
            gidx = wid * REC_PER + gi
            pltpu.sync_copy(recsrc_hbm.at[gidx], src16_v)
            pltpu.sync_copy(recdelay_hbm.at[gidx], d16_v)
            pltpu.sync_copy(recw_hbm.at[gidx], w16_v)
            flat = d16_v[...] * N + src16_v[...]
            pltpu.sync_copy(hist_hbm.at[flat], h16_v)
            val = jnp.where(valid, w16_v[...] * h16_v[...], 0.0)
            plsc.addupdate_scatter(acc_v, [(iota << 8) | dl], val)
            return 0

        lax.fori_loop(0, ngroups, g_body, 0)

    n_ff = scan_phase(ffdst_hbm, wid * FF_PER, FF_CHUNKS)
    ff_groups(n_ff)
    n_rec = scan_phase(recdst_hbm, wid * REC_PER, REC_CHUNKS)
    rec_groups(n_rec)

    def red_body(j, _):
        def inner(l, s):
            return s + acc_v[pl.ds(l * N_OUT + j * L, L)]
        outbuf_v[pl.ds(j * L, L)] = lax.fori_loop(
            0, L, inner, jnp.zeros((L,), jnp.float32))
        return 0
    lax.fori_loop(0, N_OUT // L, red_body, 0)
    pltpu.sync_copy(outbuf_v, out_hbm.at[wid])


def _tc_combine(p_ref, b_ref, o_ref):
    o_ref[:, :] = jnp.tanh(b_ref[:, :] +
                           jnp.sum(p_ref[:, :], axis=0, keepdims=True))


def kernel(obs, h_state, hist, bias, ff_w, rec_w,
           ff_src, ff_dst, rec_src, rec_dst, rec_delay):
    h_prev = jnp.concatenate([obs, h_state[N_IN:]])
    hist_flat = hist.reshape(-1)
    partials = _sc_partials(h_prev, hist_flat, ff_src, ff_dst, ff_w,
                            rec_src, rec_dst, rec_delay, rec_w)
    bias_tail = bias[N - N_OUT:].reshape(1, N_OUT)
    out = pl.pallas_call(
        _tc_combine,
        out_shape=jax.ShapeDtypeStruct((1, N_OUT), jnp.float32),
    )(partials, bias_tail)
    return out.reshape(N_OUT)


# speedup vs baseline: 479.5747x; 2.1237x over previous
import functools
import jax
import jax.numpy as jnp
from jax import lax
from jax.experimental import pallas as pl
from jax.experimental.pallas import tpu as pltpu
from jax.experimental.pallas import tpu_sc as plsc

N = 100000
N_IN = 512
N_OUT = 256
E_FF = 6400000
E_REC = 640000
THRESH = N - N_OUT

NW = 32
FF_PER = E_FF // NW
REC_PER = E_REC // NW
C_FF = 8000
C_REC = 2000
M = 8192
L = 16
CAP = M // L
NB = 8


@functools.partial(
    pl.kernel,
    mesh=plsc.VectorSubcoreMesh(core_axis_name="c", subcore_axis_name="s"),
    compiler_params=pltpu.CompilerParams(needs_layout_passes=False),
    out_type=jax.ShapeDtypeStruct((NW, N_OUT), jnp.float32),
    scratch_types=[
        pltpu.VMEM((N,), jnp.float32),
        pltpu.VMEM((C_FF,), jnp.int32),
        pltpu.VMEM((C_FF,), jnp.int32),
        pltpu.VMEM((M,), jnp.int32),
        pltpu.VMEM((L * N_OUT,), jnp.float32),
        pltpu.VMEM((NB * L,), jnp.int32),
        pltpu.VMEM((NB * L,), jnp.float32),
        pltpu.VMEM((L,), jnp.int32),
        pltpu.VMEM((L,), jnp.float32),
        pltpu.VMEM((N_OUT,), jnp.float32),
        pltpu.SemaphoreType.DMA,
        pltpu.SemaphoreType.DMA,
        pltpu.SemaphoreType.DMA,
        pltpu.SemaphoreType.DMA,
        pltpu.SemaphoreType.DMA,
        pltpu.SemaphoreType.DMA,
        pltpu.SemaphoreType.DMA,
        pltpu.SemaphoreType.DMA,
        pltpu.SemaphoreType.DMA,
        pltpu.SemaphoreType.DMA,
        pltpu.SemaphoreType.DMA,
    ],
)
def _sc_partials(hprev_hbm, hist_hbm, ffsrc_hbm, ffdst_hbm, ffw_hbm,
                 recsrc_hbm, recdst_hbm, recdelay_hbm, recw_hbm,
                 out_hbm,
                 hprev_v, buf0_v, buf1_v, enc_v, acc_v, srcg_v, wg_v,
                 d16_v, h16_v, outbuf_v,
                 sem_s0, sem_s1, sem_h,
                 sg0, sg1, sg2, sg3, sg4, sg5, sg6, sg7):
    wid = lax.axis_index("s") * 2 + lax.axis_index("c")
    iota = lax.iota(jnp.int32, L)
    lane_base = iota * CAP
    bufs = (buf0_v, buf1_v)
    ssems = (sem_s0, sem_s1)
    gsems = (sg0, sg1, sg2, sg3, sg4, sg5, sg6, sg7)

    hdesc = pltpu.async_copy(hprev_hbm, hprev_v, sem_h)

    def zacc(i, _):
        acc_v[pl.ds(i * L, L)] = jnp.zeros((L,), jnp.float32)
        return 0
    lax.fori_loop(0, (L * N_OUT) // L, zacc, 0)

    def stream_scan(dst_hbm, base, C_, nchunks, unroll):
        last = nchunks - 1

        def issue(c, b):
            pltpu.async_copy(dst_hbm.at[pl.ds(base + c * C_, C_)],
                             bufs[b].at[pl.ds(0, C_)], ssems[b])

        def wait(b):
            pltpu.make_async_copy(dst_hbm.at[pl.ds(0, C_)],
                                  bufs[b].at[pl.ds(0, C_)], ssems[b]).wait()

        def scan_buf(b, c, n_vec):
            def vb(i, carry):
                n_vec, gi = carry
                for u in range(unroll):
                    off = i * (L * unroll) + u * L
                    d = bufs[b][pl.ds(off, L)]
                    m = d >= THRESH
                    pos = lane_base + jnp.minimum(n_vec, CAP - 1)
                    enc = (gi << 8) | (d - THRESH)
                    plsc.store_scatter(enc_v, [pos], enc, mask=m)
                    n_vec = n_vec + m.astype(jnp.int32)
                    gi = gi + L
                return (n_vec, gi)
            n_vec, _ = lax.fori_loop(0, C_ // (L * unroll), vb,
                                     (n_vec, c * C_ + iota))
            return n_vec

        issue(0, 0)
        issue(1, 1)

        def pair_body(k, n_vec):
            c0 = 2 * k
            wait(0)
            n_vec = scan_buf(0, c0, n_vec)
            issue(jnp.minimum(c0 + 2, last), 0)
            wait(1)
            n_vec = scan_buf(1, c0 + 1, n_vec)
            issue(jnp.minimum(c0 + 3, last), 1)
            return n_vec

        n_vec = lax.fori_loop(0, nchunks // 2, pair_body,
                              jnp.zeros((L,), jnp.int32))
        if nchunks % 2:
            wait(0)
            n_vec = scan_buf(0, last, n_vec)
            wait(1)
        else:
            wait(0)
            wait(1)
        return jnp.minimum(n_vec, CAP)

    def e_at(r):
        return plsc.load_gather(enc_v, [lane_base + jnp.minimum(r, CAP - 1)])

    def ff_groups(n_vec):
        rmax = jnp.max(n_vec)

        def issue_g(r, p):
            e = e_at(r)
            gi = jnp.where(r < n_vec, e >> 8, 0)
            gidx = wid * FF_PER + gi
            pltpu.async_copy(ffsrc_hbm.at[gidx],
                             srcg_v.at[pl.ds(p * L, L)], gsems[p])
            pltpu.async_copy(ffw_hbm.at[gidx],
                             wg_v.at[pl.ds(p * L, L)], gsems[p])

        def wait_g(p):
            pltpu.make_async_copy(ffsrc_hbm.at[pl.ds(0, L)],
                                  srcg_v.at[pl.ds(p * L, L)], gsems[p]).wait()
            pltpu.make_async_copy(ffw_hbm.at[pl.ds(0, L)],
                                  wg_v.at[pl.ds(p * L, L)], gsems[p]).wait()

        for p in range(NB):
            issue_g(jnp.int32(p), p)

        def k_body(k, _):
            for p in range(NB):
                r = k * NB + p
                wait_g(p)
                e = e_at(r)
                valid = r < n_vec
                dl = e & (N_OUT - 1)
                h = plsc.load_gather(hprev_v, [srcg_v[pl.ds(p * L, L)]])
                val = jnp.where(valid, wg_v[pl.ds(p * L, L)] * h, 0.0)
                plsc.addupdate_scatter(acc_v, [(iota << 8) | dl], val)
                issue_g(r + NB, p)
            return 0

        lax.fori_loop(0, (rmax + NB - 1) >> 3, k_body, 0)
        for p in range(NB):
            wait_g(p)

    def rec_groups(n_vec):
        rmax = jnp.max(n_vec)

        def r_body(r, _):
            e = e_at(r)
            valid = r < n_vec
            gi = jnp.where(valid, e >> 8, 0)
            dl = e & (N_OUT - 1)
            gidx = wid * REC_PER + gi
            pltpu.async_copy(recsrc_hbm.at[gidx],
                             srcg_v.at[pl.ds(0, L)], sg0)
            pltpu.async_copy(recw_hbm.at[gidx],
                             wg_v.at[pl.ds(0, L)], sg1)
            pltpu.async_copy(recdelay_hbm.at[gidx], d16_v, sg2)
            pltpu.make_async_copy(recsrc_hbm.at[pl.ds(0, L)],
                                  srcg_v.at[pl.ds(0, L)], sg0).wait()
            pltpu.make_async_copy(recw_hbm.at[pl.ds(0, L)],
                                  wg_v.at[pl.ds(0, L)], sg1).wait()
            pltpu.make_async_copy(recdelay_hbm.at[pl.ds(0, L)],
                                  d16_v, sg2).wait()
            flat = d16_v[...] * N + srcg_v[pl.ds(0, L)]
            pltpu.async_copy(hist_hbm.at[flat], h16_v, sg3).wait()
            val = jnp.where(valid, wg_v[pl.ds(0, L)] * h16_v[...], 0.0)
            plsc.addupdate_scatter(acc_v, [(iota << 8) | dl], val)
            return 0

        lax.fori_loop(0, rmax, r_body, 0)

    n_ff = stream_scan(ffdst_hbm, wid * FF_PER, C_FF, FF_PER // C_FF, 4)
    hdesc.wait()
    ff_groups(n_ff)
    n_rec = stream_scan(recdst_hbm, wid * REC_PER, C_REC, REC_PER // C_REC, 1)
    rec_groups(n_rec)

    def red_body(j, _):
        def inner(l, s):
            return s + acc_v[pl.ds(l * N_OUT + j * L, L)]
        outbuf_v[pl.ds(j * L, L)] = lax.fori_loop(
            0, L, inner, jnp.zeros((L,), jnp.float32))
        return 0
    lax.fori_loop(0, N_OUT // L, red_body, 0)
    pltpu.sync_copy(outbuf_v, out_hbm.at[wid])


def _tc_combine(p_ref, b_ref, o_ref):
    o_ref[:, :] = jnp.tanh(b_ref[:, :] +
                           jnp.sum(p_ref[:, :], axis=0, keepdims=True))


def kernel(obs, h_state, hist, bias, ff_w, rec_w,
           ff_src, ff_dst, rec_src, rec_dst, rec_delay):
    h_prev = jnp.concatenate([obs, h_state[N_IN:]])
    hist_flat = hist.reshape(-1)
    partials = _sc_partials(h_prev, hist_flat, ff_src, ff_dst, ff_w,
                            rec_src, rec_dst, rec_delay, rec_w)
    bias_tail = bias[N - N_OUT:].reshape(1, N_OUT)
    out = pl.pallas_call(
        _tc_combine,
        out_shape=jax.ShapeDtypeStruct((1, N_OUT), jnp.float32),
    )(partials, bias_tail)
    return out.reshape(N_OUT)


# speedup vs baseline: 498.1946x; 1.0388x over previous
import functools
import jax
import jax.numpy as jnp
from jax import lax
from jax.experimental import pallas as pl
from jax.experimental.pallas import tpu as pltpu
from jax.experimental.pallas import tpu_sc as plsc

N = 100000
N_IN = 512
N_OUT = 256
E_FF = 6400000
E_REC = 640000
THRESH = N - N_OUT

NW = 32
FF_PER = E_FF // NW
REC_PER = E_REC // NW
C_FF = 8000
C_REC = 2000
L = 16
CAP = 512
M_ALLOC = L * CAP + C_FF // L + 128
NB = 8
NR = 4
UNROLL = 5


@functools.partial(
    pl.kernel,
    mesh=plsc.VectorSubcoreMesh(core_axis_name="c", subcore_axis_name="s"),
    compiler_params=pltpu.CompilerParams(needs_layout_passes=False),
    out_type=jax.ShapeDtypeStruct((NW, N_OUT), jnp.float32),
    scratch_types=[
        pltpu.VMEM((N,), jnp.float32),
        pltpu.VMEM((C_FF,), jnp.int32),
        pltpu.VMEM((C_FF,), jnp.int32),
        pltpu.VMEM((M_ALLOC,), jnp.int32),
        pltpu.VMEM((L * N_OUT,), jnp.float32),
        pltpu.VMEM((NB * L,), jnp.int32),
        pltpu.VMEM((NB * L,), jnp.float32),
        pltpu.VMEM((NR * L,), jnp.int32),
        pltpu.VMEM((NR * L,), jnp.float32),
        pltpu.VMEM((N_OUT,), jnp.float32),
        pltpu.SemaphoreType.DMA,
        pltpu.SemaphoreType.DMA,
        pltpu.SemaphoreType.DMA,
        pltpu.SemaphoreType.DMA,
        pltpu.SemaphoreType.DMA,
        pltpu.SemaphoreType.DMA,
        pltpu.SemaphoreType.DMA,
        pltpu.SemaphoreType.DMA,
        pltpu.SemaphoreType.DMA,
        pltpu.SemaphoreType.DMA,
        pltpu.SemaphoreType.DMA,
    ],
)
def _sc_partials(hprev_hbm, hist_hbm, ffsrc_hbm, ffdst_hbm, ffw_hbm,
                 recsrc_hbm, recdst_hbm, recdelay_hbm, recw_hbm,
                 out_hbm,
                 hprev_v, buf0_v, buf1_v, enc_v, acc_v, srcg_v, wg_v,
                 d16_v, h16_v, outbuf_v,
                 sem_s0, sem_s1, sem_h,
                 sg0, sg1, sg2, sg3, sg4, sg5, sg6, sg7):
    wid = lax.axis_index("s") * 2 + lax.axis_index("c")
    iota = lax.iota(jnp.int32, L)
    lane_base = iota * CAP
    bufs = (buf0_v, buf1_v)
    ssems = (sem_s0, sem_s1)
    gsems = (sg0, sg1, sg2, sg3, sg4, sg5, sg6, sg7)

    hdesc = pltpu.async_copy(hprev_hbm, hprev_v, sem_h)

    def zacc(i, _):
        acc_v[pl.ds(i * L, L)] = jnp.zeros((L,), jnp.float32)
        return 0
    lax.fori_loop(0, (L * N_OUT) // L, zacc, 0)

    def stream_scan(dst_hbm, base, C_, nchunks):
        last = nchunks - 1

        def issue(c, b):
            pltpu.async_copy(dst_hbm.at[pl.ds(base + c * C_, C_)],
                             bufs[b].at[pl.ds(0, C_)], ssems[b])

        def wait(b):
            pltpu.make_async_copy(dst_hbm.at[pl.ds(0, C_)],
                                  bufs[b].at[pl.ds(0, C_)], ssems[b]).wait()

        def scan_buf(b, c, posv):
            posv = jnp.minimum(posv, lane_base + CAP)
            encbase0 = ((c * C_ + iota) << 8) - THRESH

            def vb(i, carry):
                posv, encbase = carry
                for u in range(UNROLL):
                    off = i * (L * UNROLL) + u * L
                    d = bufs[b][pl.ds(off, L)]
                    m = d >= THRESH
                    plsc.store_scatter(enc_v, [posv], d + encbase, mask=m)
                    posv = posv + m.astype(jnp.int32)
                    encbase = encbase + (L << 8)
                return (posv, encbase)

            posv, _ = lax.fori_loop(0, C_ // (L * UNROLL), vb,
                                    (posv, encbase0))
            return posv

        issue(0, 0)
        issue(1, 1)

        def pair_body(k, posv):
            c0 = 2 * k
            wait(0)
            posv = scan_buf(0, c0, posv)
            issue(jnp.minimum(c0 + 2, last), 0)
            wait(1)
            posv = scan_buf(1, c0 + 1, posv)
            issue(jnp.minimum(c0 + 3, last), 1)
            return posv

        posv = lax.fori_loop(0, nchunks // 2, pair_body, lane_base)
        if nchunks % 2:
            wait(0)
            posv = scan_buf(0, last, posv)
            wait(1)
        else:
            wait(0)
            wait(1)
        return jnp.minimum(posv - lane_base, CAP)

    def e_at(r):
        return plsc.load_gather(enc_v, [lane_base + jnp.minimum(r, CAP - 1)])

    def ff_groups(n_vec):
        rmax = jnp.max(n_vec)

        def issue_g(r, p):
            e = e_at(r)
            gi = jnp.where(r < n_vec, e >> 8, 0)
            gidx = wid * FF_PER + gi
            pltpu.async_copy(ffsrc_hbm.at[gidx],
                             srcg_v.at[pl.ds(p * L, L)], gsems[p])
            pltpu.async_copy(ffw_hbm.at[gidx],
                             wg_v.at[pl.ds(p * L, L)], gsems[p])

        def wait_g(p):
            pltpu.make_async_copy(ffsrc_hbm.at[pl.ds(0, L)],
                                  srcg_v.at[pl.ds(p * L, L)], gsems[p]).wait()
            pltpu.make_async_copy(ffw_hbm.at[pl.ds(0, L)],
                                  wg_v.at[pl.ds(p * L, L)], gsems[p]).wait()

        for p in range(NB):
            issue_g(jnp.int32(p), p)

        def k_body(k, _):
            for p in range(NB):
                r = k * NB + p
                wait_g(p)
                e = e_at(r)
                valid = r < n_vec
                dl = e & (N_OUT - 1)
                h = plsc.load_gather(hprev_v, [srcg_v[pl.ds(p * L, L)]])
                val = jnp.where(valid, wg_v[pl.ds(p * L, L)] * h, 0.0)
                plsc.addupdate_scatter(acc_v, [(iota << 8) | dl], val)
                issue_g(r + NB, p)
            return 0

        lax.fori_loop(0, (rmax + NB - 1) >> 3, k_body, 0)
        for p in range(NB):
            wait_g(p)

    def rec_groups(n_vec):
        rmax = jnp.max(n_vec)
        asems = (sg0, sg1, sg2, sg3)
        bsems = (sg4, sg5, sg6, sg7)

        def issue_a(r, p):
            e = e_at(r)
            gi = jnp.where(r < n_vec, e >> 8, 0)
            gidx = wid * REC_PER + gi
            pltpu.async_copy(recsrc_hbm.at[gidx],
                             srcg_v.at[pl.ds(p * L, L)], asems[p])
            pltpu.async_copy(recw_hbm.at[gidx],
                             wg_v.at[pl.ds(p * L, L)], asems[p])
            pltpu.async_copy(recdelay_hbm.at[gidx],
                             d16_v.at[pl.ds(p * L, L)], asems[p])

        def wait_a(p):
            pltpu.make_async_copy(recsrc_hbm.at[pl.ds(0, L)],
                                  srcg_v.at[pl.ds(p * L, L)], asems[p]).wait()
            pltpu.make_async_copy(recw_hbm.at[pl.ds(0, L)],
                                  wg_v.at[pl.ds(p * L, L)], asems[p]).wait()
            pltpu.make_async_copy(recdelay_hbm.at[pl.ds(0, L)],
                                  d16_v.at[pl.ds(p * L, L)], asems[p]).wait()

        def issue_b(p):
            flat = d16_v[pl.ds(p * L, L)] * N + srcg_v[pl.ds(p * L, L)]
            pltpu.async_copy(hist_hbm.at[flat],
                             h16_v.at[pl.ds(p * L, L)], bsems[p])

        def wait_b(p):
            pltpu.make_async_copy(hist_hbm.at[pl.ds(0, L)],
                                  h16_v.at[pl.ds(p * L, L)], bsems[p]).wait()

        def compute(r, p):
            e = e_at(r)
            valid = r < n_vec
            dl = e & (N_OUT - 1)
            val = jnp.where(valid,
                            wg_v[pl.ds(p * L, L)] * h16_v[pl.ds(p * L, L)],
                            0.0)
            plsc.addupdate_scatter(acc_v, [(iota << 8) | dl], val)

        for p in range(NR):
            issue_a(jnp.int32(p), p)
        wait_a(0)
        issue_b(0)

        def k_body(k, _):
            for p in range(NR):
                r = k * NR + p
                if p + 1 < NR:
                    wait_a(p + 1)
                    issue_b(p + 1)
                wait_b(p)
                compute(r, p)
                issue_a(r + NR, p)
            wait_a(0)
            issue_b(0)
            return 0

        lax.fori_loop(0, (rmax + NR - 1) >> 2, k_body, 0)
        wait_b(0)
        for p in range(1, NR):
            wait_a(p)

    n_ff = stream_scan(ffdst_hbm, wid * FF_PER, C_FF, FF_PER // C_FF)
    hdesc.wait()
    ff_groups(n_ff)
    n_rec = stream_scan(recdst_hbm, wid * REC_PER, C_REC, REC_PER // C_REC)
    rec_groups(n_rec)

    def red_body(j, _):
        def inner(l, s):
            return s + acc_v[pl.ds(l * N_OUT + j * L, L)]
        outbuf_v[pl.ds(j * L, L)] = lax.fori_loop(
            0, L, inner, jnp.zeros((L,), jnp.float32))
        return 0
    lax.fori_loop(0, N_OUT // L, red_body, 0)
    pltpu.sync_copy(outbuf_v, out_hbm.at[wid])


def _tc_combine(p_ref, b_ref, o_ref):
    o_ref[:, :] = jnp.tanh(b_ref[:, :] +
                           jnp.sum(p_ref[:, :], axis=0, keepdims=True))


def kernel(obs, h_state, hist, bias, ff_w, rec_w,
           ff_src, ff_dst, rec_src, rec_dst, rec_delay):
    h_prev = jnp.concatenate([obs, h_state[N_IN:]])
    hist_flat = hist.reshape(-1)
    partials = _sc_partials(h_prev, hist_flat, ff_src, ff_dst, ff_w,
                            rec_src, rec_dst, rec_delay, rec_w)
    bias_tail = bias[N - N_OUT:].reshape(1, N_OUT)
    out = pl.pallas_call(
        _tc_combine,
        out_shape=jax.ShapeDtypeStruct((1, N_OUT), jnp.float32),
    )(partials, bias_tail)
    return out.reshape(N_OUT)
